# Initial kernel scaffold; baseline (speedup 1.0000x reference)
import jax
import jax.numpy as jnp
from jax.experimental import pallas as pl


def kernel(node_features, edge_features, We1, be1, We2, be2, ge, gbe, Wn1, bn1, Wn2, bn2, gn, gbn, edge_index):
    raise NotImplementedError("write your pallas kernel here")


# speedup vs baseline: 1.3949x; 1.3949x over previous
import functools

import jax
import jax.numpy as jnp
from jax import lax
from jax.experimental import pallas as pl
from jax.experimental.pallas import tpu as pltpu
from jax.experimental.pallas import tpu_sc as plsc

_NC = 2
_NS = 16
_NW = _NC * _NS
_LANES = 128


def _sc_gather(table, idx2d, k_group):
    _, d = table.shape
    rows = idx2d.shape[0]
    per_w = rows // _NW
    assert per_w % 8 == 0 and 8 % k_group == 0
    n_iter = per_w // 8
    sub = 8 // k_group
    mesh = plsc.VectorSubcoreMesh(core_axis_name="c", subcore_axis_name="s")

    @functools.partial(
        pl.kernel,
        out_type=jax.ShapeDtypeStruct((rows, _LANES, d), table.dtype),
        mesh=mesh,
        scratch_types=[
            pltpu.VMEM((8, _LANES), jnp.int32),
            pltpu.VMEM((k_group, _LANES, d), table.dtype),
            pltpu.SemaphoreType.DMA,
        ],
    )
    def gather_k(table_hbm, idx_hbm, out_hbm, iv, rows_v, sem):
        wid = lax.axis_index("s") * _NC + lax.axis_index("c")

        def body(t, carry):
            base = wid * per_w + t * 8
            pltpu.sync_copy(idx_hbm.at[pl.ds(base, 8)], iv)
            for half in range(sub):
                cps = [
                    pltpu.async_copy(
                        table_hbm.at[iv.at[half * k_group + j]],
                        rows_v.at[j], sem)
                    for j in range(k_group)
                ]
                for cp in cps:
                    cp.wait()
                pltpu.sync_copy(
                    rows_v, out_hbm.at[pl.ds(base + half * k_group, k_group)])
            return carry

        lax.fori_loop(0, n_iter, body, 0)

    return gather_k(table, idx2d)


def _sc_scatter(vals2, idx1, num_seg, zeros_init, k_group):
    e_pad, d = vals2.shape
    del k_group
    chunk = _LANES
    per_w = e_pad // _NW
    assert per_w % chunk == 0
    n_iter = per_w // chunk
    assert num_seg % (_NS * 8) == 0
    rows_per_tile = num_seg // _NS
    mesh = plsc.VectorSubcoreMesh(core_axis_name="c", subcore_axis_name="s")

    @functools.partial(
        pl.kernel,
        out_type=jax.ShapeDtypeStruct((_NC, num_seg, d), jnp.float32),
        mesh=mesh,
        scratch_types=[
            pltpu.VMEM((chunk,), jnp.int32),
            pltpu.VMEM((chunk, d), jnp.float32),
            pltpu.VMEM_SHARED((num_seg + 8, d), jnp.float32),
            pltpu.SemaphoreType.DMA,
        ],
    )
    def scatter_k(vals_hbm, idx_hbm, zero_hbm, out_hbm, iv, ev, acc_sh, sem):
        c = lax.axis_index("c")
        s = lax.axis_index("s")
        wid = s * _NC + c
        pltpu.sync_copy(zero_hbm,
                        acc_sh.at[pl.ds(s * rows_per_tile, rows_per_tile)])
        plsc.subcore_barrier()

        def body(t, carry):
            base = (wid * n_iter + t) * chunk
            pltpu.sync_copy(idx_hbm.at[pl.ds(base, chunk)], iv)
            pltpu.sync_copy(vals_hbm.at[pl.ds(base, chunk)], ev)
            pltpu.sync_copy(ev, acc_sh.at[iv], add=True)
            return carry

        lax.fori_loop(0, n_iter, body, 0)
        plsc.subcore_barrier()
        pltpu.sync_copy(acc_sh.at[pl.ds(s * rows_per_tile, rows_per_tile)],
                        out_hbm.at[c, pl.ds(s * rows_per_tile, rows_per_tile)])

    return scatter_k(vals2, idx1, zeros_init)


def _edge_mlp(gs, gd, e, wa, wb, wc, b1, w2, b2, g, bb, block_e):
    e_pad, ew = e.shape
    nd = gs.shape[1]
    d_e = wc.shape[0]
    hd = wa.shape[1]
    grid = e_pad // block_e

    def body(gs_r, gd_r, e_r, wa_r, wb_r, wc_r, b1_r, w2_r, b2_r, g_r, bb_r, out_r):
        eb = e_r[...][:, :d_e]
        acc = jnp.dot(gs_r[...], wa_r[...], preferred_element_type=jnp.float32)
        acc = acc + jnp.dot(gd_r[...], wb_r[...], preferred_element_type=jnp.float32)
        acc = acc + jnp.dot(eb, wc_r[...], preferred_element_type=jnp.float32)
        h = jnp.maximum(acc + b1_r[...], 0.0)
        t = jnp.dot(h, w2_r[...], preferred_element_type=jnp.float32) + b2_r[...]
        m = jnp.mean(t, axis=-1, keepdims=True)
        v = jnp.mean((t - m) * (t - m), axis=-1, keepdims=True)
        res = eb + (t - m) * lax.rsqrt(v + 1e-5) * g_r[...] + bb_r[...]
        out_r[...] = jnp.concatenate(
            [res, jnp.zeros((res.shape[0], ew - d_e), jnp.float32)], axis=-1)

    return pl.pallas_call(
        body,
        grid=(grid,),
        in_specs=[
            pl.BlockSpec((block_e, nd), lambda i: (i, 0)),
            pl.BlockSpec((block_e, nd), lambda i: (i, 0)),
            pl.BlockSpec((block_e, ew), lambda i: (i, 0)),
            pl.BlockSpec((nd, hd), lambda i: (0, 0)),
            pl.BlockSpec((nd, hd), lambda i: (0, 0)),
            pl.BlockSpec((d_e, hd), lambda i: (0, 0)),
            pl.BlockSpec((1, hd), lambda i: (0, 0)),
            pl.BlockSpec((hd, d_e), lambda i: (0, 0)),
            pl.BlockSpec((1, d_e), lambda i: (0, 0)),
            pl.BlockSpec((1, d_e), lambda i: (0, 0)),
            pl.BlockSpec((1, d_e), lambda i: (0, 0)),
        ],
        out_specs=pl.BlockSpec((block_e, ew), lambda i: (i, 0)),
        out_shape=jax.ShapeDtypeStruct((e_pad, ew), jnp.float32),
    )(gs, gd, e, wa, wb, wc, b1, w2, b2, g, bb)


def _node_mlp(x, agg2, wx, wa, b1, w2, b2, g, bb, block_n):
    n, nd = x.shape
    d_e = wa.shape[0]
    ew = agg2.shape[2]
    hd = wx.shape[1]
    grid = n // block_n

    def body(x_r, a0_r, a1_r, wx_r, wa_r, b1_r, w2_r, b2_r, g_r, bb_r, out_r):
        xb = x_r[...]
        agg = (a0_r[0] + a1_r[0])[:, :d_e]
        acc = jnp.dot(xb, wx_r[...], preferred_element_type=jnp.float32)
        acc = acc + jnp.dot(agg, wa_r[...], preferred_element_type=jnp.float32)
        h = jnp.maximum(acc + b1_r[...], 0.0)
        t = jnp.dot(h, w2_r[...], preferred_element_type=jnp.float32) + b2_r[...]
        m = jnp.mean(t, axis=-1, keepdims=True)
        v = jnp.mean((t - m) * (t - m), axis=-1, keepdims=True)
        out_r[...] = xb + (t - m) * lax.rsqrt(v + 1e-5) * g_r[...] + bb_r[...]

    return pl.pallas_call(
        body,
        grid=(grid,),
        in_specs=[
            pl.BlockSpec((block_n, nd), lambda i: (i, 0)),
            pl.BlockSpec((1, block_n, ew), lambda i: (0, i, 0)),
            pl.BlockSpec((1, block_n, ew), lambda i: (1, i, 0)),
            pl.BlockSpec((nd, hd), lambda i: (0, 0)),
            pl.BlockSpec((d_e, hd), lambda i: (0, 0)),
            pl.BlockSpec((1, hd), lambda i: (0, 0)),
            pl.BlockSpec((hd, nd), lambda i: (0, 0)),
            pl.BlockSpec((1, nd), lambda i: (0, 0)),
            pl.BlockSpec((1, nd), lambda i: (0, 0)),
            pl.BlockSpec((1, nd), lambda i: (0, 0)),
        ],
        out_specs=pl.BlockSpec((block_n, nd), lambda i: (i, 0)),
        out_shape=jax.ShapeDtypeStruct((n, nd), jnp.float32),
    )(x, agg2, agg2, wx, wa, b1, w2, b2, g, bb)


def kernel(node_features, edge_features, We1, be1, We2, be2, ge, gbe,
           Wn1, bn1, Wn2, bn2, gn, gbn, edge_index):
    n, nd = node_features.shape
    e_cnt, d_e = edge_features.shape
    n_layers, _, hd = We1.shape

    k_gather = 4
    k_scatter = 4
    group = _LANES * _NW * 8
    e_pad = -(-e_cnt // group) * group
    rows_e = e_pad // _LANES
    assert rows_e % (_NW * 8) == 0
    block_e = 1024
    assert e_pad % block_e == 0
    block_n = 1000
    assert n % block_n == 0 and n % _NS == 0

    n_pad = -(-n // (_NS * 8)) * (_NS * 8)
    pad = e_pad - e_cnt
    src_p = jnp.concatenate(
        [edge_index[0], jnp.zeros((pad,), jnp.int32)]).reshape(rows_e, _LANES)
    dst_p = jnp.concatenate(
        [edge_index[1], jnp.zeros((pad,), jnp.int32)]).reshape(rows_e, _LANES)
    dst_s = jnp.concatenate(
        [edge_index[1], jnp.full((pad,), n_pad, jnp.int32)])
    ew = 128
    e = jnp.zeros((e_pad, ew), jnp.float32).at[:e_cnt, :d_e].set(edge_features)
    zeros_init = jnp.zeros((n_pad // _NS, ew), jnp.float32)

    x = node_features
    for l in range(n_layers):
        wa = We1[l, :nd]
        wb = We1[l, nd:2 * nd]
        wc = We1[l, 2 * nd:]
        gs = _sc_gather(x, src_p, k_gather).reshape(e_pad, nd)
        gd = _sc_gather(x, dst_p, k_gather).reshape(e_pad, nd)
        e = _edge_mlp(gs, gd, e, wa, wb, wc,
                      be1[l].reshape(1, hd), We2[l], be2[l].reshape(1, d_e),
                      ge[l].reshape(1, d_e), gbe[l].reshape(1, d_e), block_e)
        agg2 = _sc_scatter(e, dst_s, n_pad, zeros_init, k_scatter)
        x = _node_mlp(x, agg2, Wn1[l, :nd], Wn1[l, nd:],
                      bn1[l].reshape(1, hd), Wn2[l], bn2[l].reshape(1, nd),
                      gn[l].reshape(1, nd), gbn[l].reshape(1, nd), block_n)
    return x
